# baseline (speedup 1.0000x reference)
import functools

import jax
import jax.numpy as jnp
from jax import lax
from jax.experimental import pallas as pl
from jax.experimental.pallas import tpu as pltpu
from jax.experimental.pallas import tpu_sc as plsc

_N = 10000
_E = 320000
_D = 128
_NPAD = 10240
_BUCK = 320
_NW = 32
_EPT = 10112
_EPAD = _NW * _EPT
_CAP = 10752
_PK = 16384
_BND = 48
_HR = 672

_mesh = plsc.VectorSubcoreMesh(core_axis_name="c", subcore_axis_name="s")

_LANE0F = None


def _lane0(dtype):
    return (lax.iota(jnp.int32, 16) == 0).astype(dtype)


@functools.partial(
    pl.kernel,
    out_type=(
        jax.ShapeDtypeStruct((_NW * _CAP,), jnp.int32),
        jax.ShapeDtypeStruct((_NW * _BND,), jnp.int32),
        jax.ShapeDtypeStruct((_NW * _NPAD,), jnp.float32),
    ),
    mesh=_mesh,
    scratch_types=[
        pltpu.VMEM((_EPT + 16,), jnp.int32),
        pltpu.VMEM((_EPT + 16,), jnp.int32),
        pltpu.VMEM((80, 128), jnp.float32),
        pltpu.VMEM((80, 128), jnp.float32),
        pltpu.VMEM((84, 128), jnp.int32),
        pltpu.VMEM((_BND + 16,), jnp.int32),
        pltpu.VMEM((768,), jnp.int32),
        pltpu.VMEM((768,), jnp.float32),
    ],
)
def _setup_sort(src_hbm, dst_hbm, sorted_hbm, bnd_hbm, hist_hbm,
                srcv, dstv, hist2, cnt2, outb2, bndb, stgi, stgf):
    cid = lax.axis_index("c")
    sid = lax.axis_index("s")
    wid = sid * 2 + cid
    lane = lax.iota(jnp.int32, 16)

    pltpu.sync_copy(src_hbm.at[pl.ds(pl.multiple_of(wid * _EPT, 8), _EPT)],
                    srcv.at[pl.ds(0, _EPT)])
    pltpu.sync_copy(dst_hbm.at[pl.ds(pl.multiple_of(wid * _EPT, 8), _EPT)],
                    dstv.at[pl.ds(0, _EPT)])

    zf = jnp.zeros((16,), jnp.float32)
    zi = jnp.zeros((16,), jnp.int32)

    def zrows(r, _):
        for c in range(8):
            hist2[r, pl.ds(c * 16, 16)] = zf
            cnt2[r, pl.ds(c * 16, 16)] = zf
            outb2[r % 84, pl.ds(c * 16, 16)] = zi
        return 0
    lax.fori_loop(0, 84, zrows, 0)

    def p1(e, _):
        d = dstv[pl.ds(e, 16)][0]
        row = lax.shift_right_logical(d, 7)
        col = jnp.bitwise_and(d, 112)
        oh = jnp.maximum(1 - jnp.abs(
            lane - jnp.broadcast_to(jnp.bitwise_and(d, 15), (16,))), 0
            ).astype(jnp.float32)
        hist2[row, pl.ds(col, 16)] = hist2[row, pl.ds(col, 16)] + oh
        return 0
    lax.fori_loop(0, _EPT, p1, 0)

    def hout(blk, _):
        for g in range(6):
            for c in range(8):
                stgf[pl.ds(g * 128 + c * 16, 16)] = (
                    hist2[blk * 6 + g, pl.ds(c * 16, 16)])
        pltpu.sync_copy(
            stgf,
            hist_hbm.at[pl.ds(pl.multiple_of(wid * _NPAD + blk * 768, 8), 768)])
        return 0
    lax.fori_loop(0, 13, hout, 0)
    for g in range(2):
        for c in range(8):
            stgf[pl.ds(g * 128 + c * 16, 16)] = hist2[78 + g, pl.ds(c * 16, 16)]
    pltpu.sync_copy(stgf.at[pl.ds(0, 256)],
                    hist_hbm.at[pl.ds(pl.multiple_of(wid * _NPAD + 9984, 8), 256)])

    def bucket_loop(t, s):
        s = jnp.bitwise_and(s + 7, -8)
        bndb[pl.ds(t, 16)] = jnp.broadcast_to(s, (16,))

        def slice_loop(ss, s2):
            g = t * 20 + ss
            row = lax.shift_right_logical(g, 3)
            col = jnp.bitwise_and(g, 7) * 16
            hv = hist2[row, pl.ds(col, 16)]
            v = jnp.broadcast_to(s2, (16,)).astype(jnp.float32)
            run = s2
            for j in range(16):
                v = jnp.where(lane == j,
                              jnp.broadcast_to(run, (16,)).astype(jnp.float32), v)
                run = run + hv[j].astype(jnp.int32)
            cnt2[row, pl.ds(col, 16)] = v
            return run
        return lax.fori_loop(0, 20, slice_loop, s)

    send = lax.fori_loop(0, _NW, bucket_loop, jnp.int32(0))
    bndb[pl.ds(_NW, 16)] = jnp.broadcast_to(send, (16,))

    def p2(e, _):
        sv = srcv[pl.ds(e, 16)][0]
        d = dstv[pl.ds(e, 16)][0]
        row = lax.shift_right_logical(d, 7)
        col = jnp.bitwise_and(d, 112)
        d15 = jnp.bitwise_and(d, 15)
        cv = cnt2[row, pl.ds(col, 16)]
        pos_f = cv[15]
        for j in range(15):
            pos_f = jnp.where(d15 == j, cv[j], pos_f)
        pos = pos_f.astype(jnp.int32)
        oh = jnp.maximum(1 - jnp.abs(lane - jnp.broadcast_to(d15, (16,))), 0
                         ).astype(jnp.float32)
        cnt2[row, pl.ds(col, 16)] = cv + oh
        p = (_N - sv) * _PK + d
        prow = lax.shift_right_logical(pos, 7)
        pcol = jnp.bitwise_and(pos, 112)
        p15 = jnp.bitwise_and(pos, 15)
        ohp = jnp.maximum(1 - jnp.abs(lane - jnp.broadcast_to(p15, (16,))), 0)
        ov = outb2[prow, pl.ds(pcol, 16)]
        outb2[prow, pl.ds(pcol, 16)] = (
            ov * (1 - ohp) + jnp.broadcast_to(p, (16,)) * ohp)
        return 0
    lax.fori_loop(0, _EPT, p2, 0)

    def sout(blk, _):
        for g in range(6):
            for c in range(8):
                stgi[pl.ds(g * 128 + c * 16, 16)] = (
                    outb2[blk * 6 + g, pl.ds(c * 16, 16)])
        pltpu.sync_copy(
            stgi,
            sorted_hbm.at[pl.ds(pl.multiple_of(wid * _CAP + blk * 768, 8), 768)])
        return 0
    lax.fori_loop(0, 14, sout, 0)

    pltpu.sync_copy(bndb.at[pl.ds(0, _BND)],
                    bnd_hbm.at[pl.ds(pl.multiple_of(wid * _BND, 8), _BND)])


@functools.partial(
    pl.kernel,
    out_type=jax.ShapeDtypeStruct((_NPAD, _D), jnp.float32),
    mesh=_mesh,
    scratch_types=[
        pltpu.VMEM((_BUCK + 16, _D), jnp.float32),
        pltpu.VMEM((256, _D), jnp.float32),
        pltpu.VMEM((256 + 16,), jnp.int32),
        pltpu.VMEM((128,), jnp.int32),
        pltpu.VMEM((128,), jnp.int32),
        pltpu.VMEM((_NW * _BND + 16,), jnp.int32),
        pltpu.SemaphoreType.DMA,
    ],
)
def _layer_agg(hs_hbm, sorted_hbm, bnd_hbm, out_hbm,
               acc, rows_v, pbuf, idxa, idxb, bndv, sem):
    cid = lax.axis_index("c")
    sid = lax.axis_index("s")
    wid = sid * 2 + cid
    base = wid * _BUCK

    zf = jnp.zeros((16,), jnp.float32)

    def za(r, _):
        for j in range(_D // 16):
            acc[r, pl.ds(j * 16, 16)] = zf
        return 0
    lax.fori_loop(0, _BUCK + 16, za, 0)

    pltpu.sync_copy(bnd_hbm, bndv.at[pl.ds(0, _NW * _BND)])

    def seg_loop(s, _):
        a0 = pl.multiple_of(bndv[pl.ds(s * _BND + wid, 16)][0], 8)
        a1 = bndv[pl.ds(s * _BND + wid + 1, 16)][0]
        nblk = lax.shift_right_logical(a1 - a0 + 255, 8)

        def block(b, _):
            off = a0 + b * 256
            pltpu.sync_copy(
                sorted_hbm.at[pl.ds(pl.multiple_of(s * _CAP + off, 8), 256)],
                pbuf.at[pl.ds(0, 256)])
            for g in range(8):
                p16 = pbuf[pl.ds(g * 16, 16)]
                sidx = _N - lax.shift_right_logical(p16, 14)
                idxa[pl.ds(g * 16, 16)] = jnp.minimum(jnp.maximum(sidx, 0), _N)
            for g in range(8):
                p16 = pbuf[pl.ds(128 + g * 16, 16)]
                sidx = _N - lax.shift_right_logical(p16, 14)
                idxb[pl.ds(g * 16, 16)] = jnp.minimum(jnp.maximum(sidx, 0), _N)
            ca = pltpu.async_copy(hs_hbm.at[idxa], rows_v.at[pl.ds(0, 128)], sem)
            cb = pltpu.async_copy(hs_hbm.at[idxb], rows_v.at[pl.ds(128, 128)], sem)
            ca.wait()
            cb.wait()

            def edge(e, _):
                p = pbuf[pl.ds(e, 16)][0]
                d = jnp.bitwise_and(p, _PK - 1) - base
                bad = jnp.logical_or(d < 0, d >= _BUCK)
                d2 = jnp.where(bad, _BUCK, d)
                for j in range(_D // 16):
                    acc[d2, pl.ds(j * 16, 16)] = (
                        acc[d2, pl.ds(j * 16, 16)] + rows_v[e, pl.ds(j * 16, 16)])
                return 0
            lax.fori_loop(0, 256, edge, 0)
            return 0
        lax.fori_loop(0, nblk, block, 0)
        return 0
    lax.fori_loop(0, _NW, seg_loop, 0)

    pltpu.sync_copy(acc.at[pl.ds(0, _BUCK)], out_hbm.at[pl.ds(base, _BUCK)])


def _prep_body(hist_ref, x_ref, w_ref, hs_ref, dinv_ref):
    ones = jnp.ones((_NW, 1), jnp.float32)
    deg = lax.dot_general(hist_ref[:], ones, (((0,), (0,)), ((), ())),
                          preferred_element_type=jnp.float32)
    dinv = lax.rsqrt(deg + 1.0)
    dinv_ref[:] = dinv
    h = jnp.dot(x_ref[:], w_ref[:], preferred_element_type=jnp.float32)
    hs_ref[pl.ds(0, _N), :] = h * dinv[:_N]
    hs_ref[pl.ds(_N, _NPAD - _N), :] = jnp.zeros((_NPAD - _N, _D), jnp.float32)


def _mid_body(agg_ref, hs_ref, dinv_ref, g_ref, be_ref, w_ref, out_ref):
    dinv = dinv_ref[pl.ds(0, _N)]
    t = (agg_ref[pl.ds(0, _N), :] + hs_ref[pl.ds(0, _N), :]) * dinv
    m = jnp.mean(t, axis=0, keepdims=True)
    y0 = t - m
    v = jnp.mean(y0 * y0, axis=0, keepdims=True)
    y = jnp.maximum(y0 * lax.rsqrt(v + 1e-5) * g_ref[:] + be_ref[:], 0.0)
    hn = jnp.dot(y, w_ref[:], preferred_element_type=jnp.float32) * dinv
    wout = w_ref.shape[1]
    out_ref[pl.ds(0, _N), :] = hn
    out_ref[pl.ds(_N, _NPAD - _N), :] = jnp.zeros((_NPAD - _N, wout), jnp.float32)


def _final_body(agg_ref, hs_ref, dinv_ref, g_ref, be_ref, wr_ref, br_ref, out_ref):
    dinv = dinv_ref[pl.ds(0, _N)]
    t = (agg_ref[pl.ds(0, _N), :] + hs_ref[pl.ds(0, _N), :]) * dinv
    m = jnp.mean(t, axis=0, keepdims=True)
    y0 = t - m
    v = jnp.mean(y0 * y0, axis=0, keepdims=True)
    y = jnp.maximum(y0 * lax.rsqrt(v + 1e-5) * g_ref[:] + be_ref[:], 0.0)
    z = jnp.dot(y, wr_ref[:], preferred_element_type=jnp.float32) + br_ref[:]
    out_ref[:] = jax.nn.sigmoid(z)


def kernel(x, edge_index, W1, b1, g1, be1, W2, b2, g2, be2, W3, b3, g3, be3, Wr, br):
    del b1, b2, b3
    ei = edge_index.astype(jnp.int32)
    padlen = _EPAD - _E
    pad = jnp.full((padlen,), _N, jnp.int32)
    src = jnp.concatenate([ei[0], pad])
    dst = jnp.concatenate([ei[1], pad])

    sorted_p, bnd, hist = _setup_sort(src, dst)
    hist2d = hist.reshape(_NW, _NPAD)

    hs1, dinv = pl.pallas_call(
        _prep_body,
        out_shape=(
            jax.ShapeDtypeStruct((_NPAD, _D), jnp.float32),
            jax.ShapeDtypeStruct((_NPAD, 1), jnp.float32),
        ),
    )(hist2d, x, W1)

    agg1 = _layer_agg(hs1, sorted_p, bnd)

    hs2 = pl.pallas_call(
        _mid_body,
        out_shape=jax.ShapeDtypeStruct((_NPAD, _D), jnp.float32),
    )(agg1, hs1, dinv, g1.reshape(1, -1), be1.reshape(1, -1), W2)

    agg2 = _layer_agg(hs2, sorted_p, bnd)

    half = _D // 2
    W3p = jnp.pad(W3, ((0, 0), (0, _D - half)))
    g3p = jnp.pad(g3, (0, _D - half)).reshape(1, -1)
    be3p = jnp.pad(be3, (0, _D - half)).reshape(1, -1)
    Wrp = jnp.pad(Wr, ((0, _D - half), (0, 0)))

    hs3 = pl.pallas_call(
        _mid_body,
        out_shape=jax.ShapeDtypeStruct((_NPAD, _D), jnp.float32),
    )(agg2, hs2, dinv, g2.reshape(1, -1), be2.reshape(1, -1), W3p)

    agg3 = _layer_agg(hs3, sorted_p, bnd)

    out = pl.pallas_call(
        _final_body,
        out_shape=jax.ShapeDtypeStruct((_N, 1), jnp.float32),
    )(agg3, hs3, dinv, g3p, be3p, Wrp, br.reshape(1, 1))

    return out


# speedup vs baseline: 1.2061x; 1.2061x over previous
import functools

import jax
import jax.numpy as jnp
from jax import lax
from jax.experimental import pallas as pl
from jax.experimental.pallas import tpu as pltpu
from jax.experimental.pallas import tpu_sc as plsc

_N = 10000
_E = 320000
_D = 128
_NPAD = 10240
_BUCK = 320
_NW = 32
_EPT = 10112
_EPAD = _NW * _EPT
_CAP = 10752
_PK = 16384
_BND = 48
_HR = 672

_mesh = plsc.VectorSubcoreMesh(core_axis_name="c", subcore_axis_name="s")

_LANE0F = None


def _lane0(dtype):
    return (lax.iota(jnp.int32, 16) == 0).astype(dtype)


@functools.partial(
    pl.kernel,
    out_type=(
        jax.ShapeDtypeStruct((_NW * _CAP,), jnp.int32),
        jax.ShapeDtypeStruct((_NW * _BND,), jnp.int32),
        jax.ShapeDtypeStruct((_NW * _NPAD,), jnp.float32),
    ),
    mesh=_mesh,
    scratch_types=[
        pltpu.VMEM((_EPT + 16,), jnp.int32),
        pltpu.VMEM((_EPT + 16,), jnp.int32),
        pltpu.VMEM((80, 128), jnp.float32),
        pltpu.VMEM((80, 128), jnp.float32),
        pltpu.VMEM((84, 128), jnp.int32),
        pltpu.VMEM((_BND + 16,), jnp.int32),
        pltpu.VMEM((768,), jnp.int32),
        pltpu.VMEM((768,), jnp.float32),
    ],
)
def _setup_sort(src_hbm, dst_hbm, sorted_hbm, bnd_hbm, hist_hbm,
                srcv, dstv, hist2, cnt2, outb2, bndb, stgi, stgf):
    cid = lax.axis_index("c")
    sid = lax.axis_index("s")
    wid = sid * 2 + cid
    lane = lax.iota(jnp.int32, 16)

    pltpu.sync_copy(src_hbm.at[pl.ds(pl.multiple_of(wid * _EPT, 8), _EPT)],
                    srcv.at[pl.ds(0, _EPT)])
    pltpu.sync_copy(dst_hbm.at[pl.ds(pl.multiple_of(wid * _EPT, 8), _EPT)],
                    dstv.at[pl.ds(0, _EPT)])

    zf = jnp.zeros((16,), jnp.float32)
    zi = jnp.zeros((16,), jnp.int32)

    def zrows(r, _):
        for c in range(8):
            hist2[r, pl.ds(c * 16, 16)] = zf
            cnt2[r, pl.ds(c * 16, 16)] = zf
            outb2[r % 84, pl.ds(c * 16, 16)] = zi
        return 0
    lax.fori_loop(0, 84, zrows, 0)

    def p1(e, _):
        d = dstv[pl.ds(e, 16)][0]
        row = lax.shift_right_logical(d, 7)
        col = jnp.bitwise_and(d, 112)
        oh = jnp.maximum(1 - jnp.abs(
            lane - jnp.broadcast_to(jnp.bitwise_and(d, 15), (16,))), 0
            ).astype(jnp.float32)
        hist2[row, pl.ds(col, 16)] = hist2[row, pl.ds(col, 16)] + oh
        return 0
    lax.fori_loop(0, _EPT, p1, 0)

    def hout(blk, _):
        for g in range(6):
            for c in range(8):
                stgf[pl.ds(g * 128 + c * 16, 16)] = (
                    hist2[blk * 6 + g, pl.ds(c * 16, 16)])
        pltpu.sync_copy(
            stgf,
            hist_hbm.at[pl.ds(pl.multiple_of(wid * _NPAD + blk * 768, 8), 768)])
        return 0
    lax.fori_loop(0, 13, hout, 0)
    for g in range(2):
        for c in range(8):
            stgf[pl.ds(g * 128 + c * 16, 16)] = hist2[78 + g, pl.ds(c * 16, 16)]
    pltpu.sync_copy(stgf.at[pl.ds(0, 256)],
                    hist_hbm.at[pl.ds(pl.multiple_of(wid * _NPAD + 9984, 8), 256)])

    def bucket_loop(t, s):
        s = jnp.bitwise_and(s + 7, -8)
        bndb[pl.ds(t, 16)] = jnp.broadcast_to(s, (16,))

        def slice_loop(ss, s2):
            g = t * 20 + ss
            row = lax.shift_right_logical(g, 3)
            col = jnp.bitwise_and(g, 7) * 16
            hv = hist2[row, pl.ds(col, 16)]
            v = jnp.broadcast_to(s2, (16,)).astype(jnp.float32)
            run = s2
            for j in range(16):
                v = jnp.where(lane == j,
                              jnp.broadcast_to(run, (16,)).astype(jnp.float32), v)
                run = run + hv[j].astype(jnp.int32)
            cnt2[row, pl.ds(col, 16)] = v
            return run
        return lax.fori_loop(0, 20, slice_loop, s)

    send = lax.fori_loop(0, _NW, bucket_loop, jnp.int32(0))
    bndb[pl.ds(_NW, 16)] = jnp.broadcast_to(send, (16,))

    def p2(e, _):
        sv = srcv[pl.ds(e, 16)][0]
        d = dstv[pl.ds(e, 16)][0]
        row = lax.shift_right_logical(d, 7)
        col = jnp.bitwise_and(d, 112)
        d15 = jnp.bitwise_and(d, 15)
        cv = cnt2[row, pl.ds(col, 16)]
        pos_f = cv[15]
        for j in range(15):
            pos_f = jnp.where(d15 == j, cv[j], pos_f)
        pos = pos_f.astype(jnp.int32)
        oh = jnp.maximum(1 - jnp.abs(lane - jnp.broadcast_to(d15, (16,))), 0
                         ).astype(jnp.float32)
        cnt2[row, pl.ds(col, 16)] = cv + oh
        p = (_N - sv) * _PK + d
        prow = lax.shift_right_logical(pos, 7)
        pcol = jnp.bitwise_and(pos, 112)
        p15 = jnp.bitwise_and(pos, 15)
        ohp = jnp.maximum(1 - jnp.abs(lane - jnp.broadcast_to(p15, (16,))), 0)
        ov = outb2[prow, pl.ds(pcol, 16)]
        outb2[prow, pl.ds(pcol, 16)] = (
            ov * (1 - ohp) + jnp.broadcast_to(p, (16,)) * ohp)
        return 0
    lax.fori_loop(0, _EPT, p2, 0)

    def sout(blk, _):
        for g in range(6):
            for c in range(8):
                stgi[pl.ds(g * 128 + c * 16, 16)] = (
                    outb2[blk * 6 + g, pl.ds(c * 16, 16)])
        pltpu.sync_copy(
            stgi,
            sorted_hbm.at[pl.ds(pl.multiple_of(wid * _CAP + blk * 768, 8), 768)])
        return 0
    lax.fori_loop(0, 14, sout, 0)

    pltpu.sync_copy(bndb.at[pl.ds(0, _BND)],
                    bnd_hbm.at[pl.ds(pl.multiple_of(wid * _BND, 8), _BND)])


@functools.partial(
    pl.kernel,
    out_type=jax.ShapeDtypeStruct((_NPAD, _D), jnp.float32),
    mesh=_mesh,
    scratch_types=[
        pltpu.VMEM((_BUCK + 16, _D), jnp.float32),
        pltpu.VMEM((256, _D), jnp.float32),
        pltpu.VMEM((256 + 16,), jnp.int32),
        pltpu.VMEM((128,), jnp.int32),
        pltpu.VMEM((128,), jnp.int32),
        pltpu.VMEM((_NW * _BND + 16,), jnp.int32),
        pltpu.SemaphoreType.DMA,
    ],
)
def _layer_agg(hs_hbm, sorted_hbm, bnd_hbm, out_hbm,
               acc, rows_v, pbuf, idxa, idxb, bndv, sem):
    cid = lax.axis_index("c")
    sid = lax.axis_index("s")
    wid = sid * 2 + cid
    base = wid * _BUCK

    zf = jnp.zeros((16,), jnp.float32)

    def za(r, _):
        for j in range(_D // 16):
            acc[r, pl.ds(j * 16, 16)] = zf
        return 0
    lax.fori_loop(0, _BUCK + 16, za, 0)

    pltpu.sync_copy(bnd_hbm, bndv.at[pl.ds(0, _NW * _BND)])

    def seg_loop(s, _):
        a0 = pl.multiple_of(bndv[pl.ds(s * _BND + wid, 16)][0], 8)
        a1 = bndv[pl.ds(s * _BND + wid + 1, 16)][0]
        nblk = lax.shift_right_logical(a1 - a0 + 255, 8)

        def block(b, _):
            off = a0 + b * 256
            pltpu.sync_copy(
                sorted_hbm.at[pl.ds(pl.multiple_of(s * _CAP + off, 8), 256)],
                pbuf.at[pl.ds(0, 256)])
            for g in range(8):
                p16 = pbuf[pl.ds(g * 16, 16)]
                sidx = _N - lax.shift_right_logical(p16, 14)
                idxa[pl.ds(g * 16, 16)] = jnp.minimum(jnp.maximum(sidx, 0), _N)
            for g in range(8):
                p16 = pbuf[pl.ds(128 + g * 16, 16)]
                sidx = _N - lax.shift_right_logical(p16, 14)
                idxb[pl.ds(g * 16, 16)] = jnp.minimum(jnp.maximum(sidx, 0), _N)
            ca = pltpu.async_copy(hs_hbm.at[idxa], rows_v.at[pl.ds(0, 128)], sem)
            cb = pltpu.async_copy(hs_hbm.at[idxb], rows_v.at[pl.ds(128, 128)], sem)
            ca.wait()
            cb.wait()

            nloc = jnp.minimum(a1 - off, 256)

            def edge(e, _):
                p = pbuf[pl.ds(e, 16)][0]
                d = jnp.bitwise_and(p, _PK - 1) - base
                bad = jnp.logical_or(d < 0, d >= _BUCK)
                d2 = jnp.where(bad, _BUCK, d)
                for j in range(_D // 16):
                    acc[d2, pl.ds(j * 16, 16)] = (
                        acc[d2, pl.ds(j * 16, 16)] + rows_v[e, pl.ds(j * 16, 16)])
                return 0
            lax.fori_loop(0, nloc, edge, 0)
            return 0
        lax.fori_loop(0, nblk, block, 0)
        return 0
    lax.fori_loop(0, _NW, seg_loop, 0)

    pltpu.sync_copy(acc.at[pl.ds(0, _BUCK)], out_hbm.at[pl.ds(base, _BUCK)])


def _prep_body(hist_ref, x_ref, w_ref, hs_ref, dinv_ref):
    ones = jnp.ones((_NW, 1), jnp.float32)
    deg = lax.dot_general(hist_ref[:], ones, (((0,), (0,)), ((), ())),
                          preferred_element_type=jnp.float32)
    dinv = lax.rsqrt(deg + 1.0)
    dinv_ref[:] = dinv
    h = jnp.dot(x_ref[:], w_ref[:], preferred_element_type=jnp.float32)
    hs_ref[pl.ds(0, _N), :] = h * dinv[:_N]
    hs_ref[pl.ds(_N, _NPAD - _N), :] = jnp.zeros((_NPAD - _N, _D), jnp.float32)


def _mid_body(agg_ref, hs_ref, dinv_ref, g_ref, be_ref, w_ref, out_ref):
    dinv = dinv_ref[pl.ds(0, _N)]
    t = (agg_ref[pl.ds(0, _N), :] + hs_ref[pl.ds(0, _N), :]) * dinv
    m = jnp.mean(t, axis=0, keepdims=True)
    y0 = t - m
    v = jnp.mean(y0 * y0, axis=0, keepdims=True)
    y = jnp.maximum(y0 * lax.rsqrt(v + 1e-5) * g_ref[:] + be_ref[:], 0.0)
    hn = jnp.dot(y, w_ref[:], preferred_element_type=jnp.float32) * dinv
    wout = w_ref.shape[1]
    out_ref[pl.ds(0, _N), :] = hn
    out_ref[pl.ds(_N, _NPAD - _N), :] = jnp.zeros((_NPAD - _N, wout), jnp.float32)


def _final_body(agg_ref, hs_ref, dinv_ref, g_ref, be_ref, wr_ref, br_ref, out_ref):
    dinv = dinv_ref[pl.ds(0, _N)]
    t = (agg_ref[pl.ds(0, _N), :] + hs_ref[pl.ds(0, _N), :]) * dinv
    m = jnp.mean(t, axis=0, keepdims=True)
    y0 = t - m
    v = jnp.mean(y0 * y0, axis=0, keepdims=True)
    y = jnp.maximum(y0 * lax.rsqrt(v + 1e-5) * g_ref[:] + be_ref[:], 0.0)
    z = jnp.dot(y, wr_ref[:], preferred_element_type=jnp.float32) + br_ref[:]
    out_ref[:] = jax.nn.sigmoid(z)


def kernel(x, edge_index, W1, b1, g1, be1, W2, b2, g2, be2, W3, b3, g3, be3, Wr, br):
    del b1, b2, b3
    ei = edge_index.astype(jnp.int32)
    padlen = _EPAD - _E
    pad = jnp.full((padlen,), _N, jnp.int32)
    src = jnp.concatenate([ei[0], pad])
    dst = jnp.concatenate([ei[1], pad])

    sorted_p, bnd, hist = _setup_sort(src, dst)
    hist2d = hist.reshape(_NW, _NPAD)

    hs1, dinv = pl.pallas_call(
        _prep_body,
        out_shape=(
            jax.ShapeDtypeStruct((_NPAD, _D), jnp.float32),
            jax.ShapeDtypeStruct((_NPAD, 1), jnp.float32),
        ),
    )(hist2d, x, W1)

    agg1 = _layer_agg(hs1, sorted_p, bnd)

    hs2 = pl.pallas_call(
        _mid_body,
        out_shape=jax.ShapeDtypeStruct((_NPAD, _D), jnp.float32),
    )(agg1, hs1, dinv, g1.reshape(1, -1), be1.reshape(1, -1), W2)

    agg2 = _layer_agg(hs2, sorted_p, bnd)

    half = _D // 2
    W3p = jnp.pad(W3, ((0, 0), (0, _D - half)))
    g3p = jnp.pad(g3, (0, _D - half)).reshape(1, -1)
    be3p = jnp.pad(be3, (0, _D - half)).reshape(1, -1)
    Wrp = jnp.pad(Wr, ((0, _D - half), (0, 0)))

    hs3 = pl.pallas_call(
        _mid_body,
        out_shape=jax.ShapeDtypeStruct((_NPAD, _D), jnp.float32),
    )(agg2, hs2, dinv, g2.reshape(1, -1), be2.reshape(1, -1), W3p)

    agg3 = _layer_agg(hs3, sorted_p, bnd)

    out = pl.pallas_call(
        _final_body,
        out_shape=jax.ShapeDtypeStruct((_N, 1), jnp.float32),
    )(agg3, hs3, dinv, g3p, be3p, Wrp, br.reshape(1, 1))

    return out


# speedup vs baseline: 1.4435x; 1.1969x over previous
import functools

import jax
import jax.numpy as jnp
from jax import lax
from jax.experimental import pallas as pl
from jax.experimental.pallas import tpu as pltpu
from jax.experimental.pallas import tpu_sc as plsc

_N = 10000
_E = 320000
_D = 128
_NPAD = 10240
_BUCK = 320
_NW = 32
_EPT = 10112
_EPAD = _NW * _EPT
_CAP = 10752
_PK = 16384
_BND = 48
_HR = 672

_mesh = plsc.VectorSubcoreMesh(core_axis_name="c", subcore_axis_name="s")

_LANE0F = None


def _lane0(dtype):
    return (lax.iota(jnp.int32, 16) == 0).astype(dtype)


@functools.partial(
    pl.kernel,
    out_type=(
        jax.ShapeDtypeStruct((_NW * _CAP,), jnp.int32),
        jax.ShapeDtypeStruct((_NW * _BND,), jnp.int32),
        jax.ShapeDtypeStruct((_NW * _NPAD,), jnp.float32),
    ),
    mesh=_mesh,
    scratch_types=[
        pltpu.VMEM((_EPT + 16,), jnp.int32),
        pltpu.VMEM((_EPT + 16,), jnp.int32),
        pltpu.VMEM((80, 128), jnp.float32),
        pltpu.VMEM((80, 128), jnp.float32),
        pltpu.VMEM((84, 128), jnp.int32),
        pltpu.VMEM((_BND + 16,), jnp.int32),
        pltpu.VMEM((768,), jnp.int32),
        pltpu.VMEM((768,), jnp.float32),
    ],
)
def _setup_sort(src_hbm, dst_hbm, sorted_hbm, bnd_hbm, hist_hbm,
                srcv, dstv, hist2, cnt2, outb2, bndb, stgi, stgf):
    cid = lax.axis_index("c")
    sid = lax.axis_index("s")
    wid = sid * 2 + cid
    lane = lax.iota(jnp.int32, 16)

    pltpu.sync_copy(src_hbm.at[pl.ds(pl.multiple_of(wid * _EPT, 8), _EPT)],
                    srcv.at[pl.ds(0, _EPT)])
    pltpu.sync_copy(dst_hbm.at[pl.ds(pl.multiple_of(wid * _EPT, 8), _EPT)],
                    dstv.at[pl.ds(0, _EPT)])

    zf = jnp.zeros((16,), jnp.float32)
    zi = jnp.zeros((16,), jnp.int32)

    def zrows(r, _):
        for c in range(8):
            hist2[r, pl.ds(c * 16, 16)] = zf
            cnt2[r, pl.ds(c * 16, 16)] = zf
            outb2[r % 84, pl.ds(c * 16, 16)] = zi
        return 0
    lax.fori_loop(0, 84, zrows, 0)

    def p1(e, _):
        d = dstv[pl.ds(e, 16)][0]
        row = lax.shift_right_logical(d, 7)
        col = jnp.bitwise_and(d, 112)
        oh = jnp.maximum(1 - jnp.abs(
            lane - jnp.broadcast_to(jnp.bitwise_and(d, 15), (16,))), 0
            ).astype(jnp.float32)
        hist2[row, pl.ds(col, 16)] = hist2[row, pl.ds(col, 16)] + oh
        return 0
    lax.fori_loop(0, _EPT, p1, 0)

    def hout(blk, _):
        for g in range(6):
            for c in range(8):
                stgf[pl.ds(g * 128 + c * 16, 16)] = (
                    hist2[blk * 6 + g, pl.ds(c * 16, 16)])
        pltpu.sync_copy(
            stgf,
            hist_hbm.at[pl.ds(pl.multiple_of(wid * _NPAD + blk * 768, 8), 768)])
        return 0
    lax.fori_loop(0, 13, hout, 0)
    for g in range(2):
        for c in range(8):
            stgf[pl.ds(g * 128 + c * 16, 16)] = hist2[78 + g, pl.ds(c * 16, 16)]
    pltpu.sync_copy(stgf.at[pl.ds(0, 256)],
                    hist_hbm.at[pl.ds(pl.multiple_of(wid * _NPAD + 9984, 8), 256)])

    def bucket_loop(t, s):
        s = jnp.bitwise_and(s + 7, -8)
        bndb[pl.ds(t, 16)] = jnp.broadcast_to(s, (16,))

        def slice_loop(ss, s2):
            g = t * 20 + ss
            row = lax.shift_right_logical(g, 3)
            col = jnp.bitwise_and(g, 7) * 16
            hv = hist2[row, pl.ds(col, 16)]
            v = jnp.broadcast_to(s2, (16,)).astype(jnp.float32)
            run = s2
            for j in range(16):
                v = jnp.where(lane == j,
                              jnp.broadcast_to(run, (16,)).astype(jnp.float32), v)
                run = run + hv[j].astype(jnp.int32)
            cnt2[row, pl.ds(col, 16)] = v
            return run
        return lax.fori_loop(0, 20, slice_loop, s)

    send = lax.fori_loop(0, _NW, bucket_loop, jnp.int32(0))
    bndb[pl.ds(_NW, 16)] = jnp.broadcast_to(send, (16,))

    def p2(e, _):
        sv = srcv[pl.ds(e, 16)][0]
        d = dstv[pl.ds(e, 16)][0]
        row = lax.shift_right_logical(d, 7)
        col = jnp.bitwise_and(d, 112)
        d15 = jnp.bitwise_and(d, 15)
        cv = cnt2[row, pl.ds(col, 16)]
        pos_f = cv[15]
        for j in range(15):
            pos_f = jnp.where(d15 == j, cv[j], pos_f)
        pos = pos_f.astype(jnp.int32)
        oh = jnp.maximum(1 - jnp.abs(lane - jnp.broadcast_to(d15, (16,))), 0
                         ).astype(jnp.float32)
        cnt2[row, pl.ds(col, 16)] = cv + oh
        p = (_N - sv) * _PK + d
        prow = lax.shift_right_logical(pos, 7)
        pcol = jnp.bitwise_and(pos, 112)
        p15 = jnp.bitwise_and(pos, 15)
        ohp = jnp.maximum(1 - jnp.abs(lane - jnp.broadcast_to(p15, (16,))), 0)
        ov = outb2[prow, pl.ds(pcol, 16)]
        outb2[prow, pl.ds(pcol, 16)] = (
            ov * (1 - ohp) + jnp.broadcast_to(p, (16,)) * ohp)
        return 0
    lax.fori_loop(0, _EPT, p2, 0)

    def sout(blk, _):
        for g in range(6):
            for c in range(8):
                stgi[pl.ds(g * 128 + c * 16, 16)] = (
                    outb2[blk * 6 + g, pl.ds(c * 16, 16)])
        pltpu.sync_copy(
            stgi,
            sorted_hbm.at[pl.ds(pl.multiple_of(wid * _CAP + blk * 768, 8), 768)])
        return 0
    lax.fori_loop(0, 14, sout, 0)

    pltpu.sync_copy(bndb.at[pl.ds(0, _BND)],
                    bnd_hbm.at[pl.ds(pl.multiple_of(wid * _BND, 8), _BND)])


@functools.partial(
    pl.kernel,
    out_type=jax.ShapeDtypeStruct((_NPAD, _D), jnp.float32),
    mesh=_mesh,
    scratch_types=[
        pltpu.VMEM((_BUCK + 16, _D), jnp.float32),
        pltpu.VMEM((256, _D), jnp.float32),
        pltpu.VMEM((256 + 16,), jnp.int32),
        pltpu.VMEM((128,), jnp.int32),
        pltpu.VMEM((128,), jnp.int32),
        pltpu.VMEM((_NW * _BND + 16,), jnp.int32),
        pltpu.SemaphoreType.DMA,
    ],
)
def _layer_agg(hs_hbm, sorted_hbm, bnd_hbm, out_hbm,
               acc, rows_v, pbuf, idxa, idxb, bndv, sem):
    cid = lax.axis_index("c")
    sid = lax.axis_index("s")
    wid = sid * 2 + cid
    base = wid * _BUCK

    zf = jnp.zeros((16,), jnp.float32)

    def za(r, _):
        for j in range(_D // 16):
            acc[r, pl.ds(j * 16, 16)] = zf
        return 0
    lax.fori_loop(0, _BUCK + 16, za, 0)

    pltpu.sync_copy(bnd_hbm, bndv.at[pl.ds(0, _NW * _BND)])

    def seg_loop(s, _):
        a0 = pl.multiple_of(bndv[pl.ds(s * _BND + wid, 16)][0], 8)
        a1 = bndv[pl.ds(s * _BND + wid + 1, 16)][0]
        nblk = lax.shift_right_logical(a1 - a0 + 255, 8)

        def block(b, _):
            off = a0 + b * 256
            pltpu.sync_copy(
                sorted_hbm.at[pl.ds(pl.multiple_of(s * _CAP + off, 8), 256)],
                pbuf.at[pl.ds(0, 256)])
            for g in range(8):
                p16 = pbuf[pl.ds(g * 16, 16)]
                sidx = _N - lax.shift_right_logical(p16, 14)
                idxa[pl.ds(g * 16, 16)] = jnp.minimum(jnp.maximum(sidx, 0), _N)
            nloc = jnp.minimum(a1 - off, 256)
            ca = pltpu.async_copy(hs_hbm.at[idxa], rows_v.at[pl.ds(0, 128)], sem)

            @pl.when(nloc > 128)
            def _():
                for g in range(8):
                    p16 = pbuf[pl.ds(128 + g * 16, 16)]
                    sidx = _N - lax.shift_right_logical(p16, 14)
                    idxb[pl.ds(g * 16, 16)] = jnp.minimum(jnp.maximum(sidx, 0), _N)
                cb = pltpu.async_copy(hs_hbm.at[idxb],
                                      rows_v.at[pl.ds(128, 128)], sem)
                cb.wait()
            ca.wait()

            def edge(e, _):
                p = pbuf[pl.ds(e, 16)][0]
                d = jnp.bitwise_and(p, _PK - 1) - base
                bad = jnp.logical_or(d < 0, d >= _BUCK)
                d2 = jnp.where(bad, _BUCK, d)
                for j in range(_D // 16):
                    acc[d2, pl.ds(j * 16, 16)] = (
                        acc[d2, pl.ds(j * 16, 16)] + rows_v[e, pl.ds(j * 16, 16)])
                return 0
            lax.fori_loop(0, nloc, edge, 0)
            return 0
        lax.fori_loop(0, nblk, block, 0)
        return 0
    lax.fori_loop(0, _NW, seg_loop, 0)

    pltpu.sync_copy(acc.at[pl.ds(0, _BUCK)], out_hbm.at[pl.ds(base, _BUCK)])


def _prep_body(hist_ref, x_ref, w_ref, hs_ref, dinv_ref):
    ones = jnp.ones((_NW, 1), jnp.float32)
    deg = lax.dot_general(hist_ref[:], ones, (((0,), (0,)), ((), ())),
                          preferred_element_type=jnp.float32)
    dinv = lax.rsqrt(deg + 1.0)
    dinv_ref[:] = dinv
    h = jnp.dot(x_ref[:], w_ref[:], preferred_element_type=jnp.float32)
    hs_ref[pl.ds(0, _N), :] = h * dinv[:_N]
    hs_ref[pl.ds(_N, _NPAD - _N), :] = jnp.zeros((_NPAD - _N, _D), jnp.float32)


def _mid_body(agg_ref, hs_ref, dinv_ref, g_ref, be_ref, w_ref, out_ref):
    dinv = dinv_ref[pl.ds(0, _N)]
    t = (agg_ref[pl.ds(0, _N), :] + hs_ref[pl.ds(0, _N), :]) * dinv
    m = jnp.mean(t, axis=0, keepdims=True)
    y0 = t - m
    v = jnp.mean(y0 * y0, axis=0, keepdims=True)
    y = jnp.maximum(y0 * lax.rsqrt(v + 1e-5) * g_ref[:] + be_ref[:], 0.0)
    hn = jnp.dot(y, w_ref[:], preferred_element_type=jnp.float32) * dinv
    wout = w_ref.shape[1]
    out_ref[pl.ds(0, _N), :] = hn
    out_ref[pl.ds(_N, _NPAD - _N), :] = jnp.zeros((_NPAD - _N, wout), jnp.float32)


def _final_body(agg_ref, hs_ref, dinv_ref, g_ref, be_ref, wr_ref, br_ref, out_ref):
    dinv = dinv_ref[pl.ds(0, _N)]
    t = (agg_ref[pl.ds(0, _N), :] + hs_ref[pl.ds(0, _N), :]) * dinv
    m = jnp.mean(t, axis=0, keepdims=True)
    y0 = t - m
    v = jnp.mean(y0 * y0, axis=0, keepdims=True)
    y = jnp.maximum(y0 * lax.rsqrt(v + 1e-5) * g_ref[:] + be_ref[:], 0.0)
    z = jnp.dot(y, wr_ref[:], preferred_element_type=jnp.float32) + br_ref[:]
    out_ref[:] = jax.nn.sigmoid(z)


def kernel(x, edge_index, W1, b1, g1, be1, W2, b2, g2, be2, W3, b3, g3, be3, Wr, br):
    del b1, b2, b3
    ei = edge_index.astype(jnp.int32)
    padlen = _EPAD - _E
    pad = jnp.full((padlen,), _N, jnp.int32)
    src = jnp.concatenate([ei[0], pad])
    dst = jnp.concatenate([ei[1], pad])

    sorted_p, bnd, hist = _setup_sort(src, dst)
    hist2d = hist.reshape(_NW, _NPAD)

    hs1, dinv = pl.pallas_call(
        _prep_body,
        out_shape=(
            jax.ShapeDtypeStruct((_NPAD, _D), jnp.float32),
            jax.ShapeDtypeStruct((_NPAD, 1), jnp.float32),
        ),
    )(hist2d, x, W1)

    agg1 = _layer_agg(hs1, sorted_p, bnd)

    hs2 = pl.pallas_call(
        _mid_body,
        out_shape=jax.ShapeDtypeStruct((_NPAD, _D), jnp.float32),
    )(agg1, hs1, dinv, g1.reshape(1, -1), be1.reshape(1, -1), W2)

    agg2 = _layer_agg(hs2, sorted_p, bnd)

    half = _D // 2
    W3p = jnp.pad(W3, ((0, 0), (0, _D - half)))
    g3p = jnp.pad(g3, (0, _D - half)).reshape(1, -1)
    be3p = jnp.pad(be3, (0, _D - half)).reshape(1, -1)
    Wrp = jnp.pad(Wr, ((0, _D - half), (0, 0)))

    hs3 = pl.pallas_call(
        _mid_body,
        out_shape=jax.ShapeDtypeStruct((_NPAD, _D), jnp.float32),
    )(agg2, hs2, dinv, g2.reshape(1, -1), be2.reshape(1, -1), W3p)

    agg3 = _layer_agg(hs3, sorted_p, bnd)

    out = pl.pallas_call(
        _final_body,
        out_shape=jax.ShapeDtypeStruct((_N, 1), jnp.float32),
    )(agg3, hs3, dinv, g3p, be3p, Wrp, br.reshape(1, 1))

    return out


# speedup vs baseline: 1.5374x; 1.0651x over previous
import functools

import jax
import jax.numpy as jnp
from jax import lax
from jax.experimental import pallas as pl
from jax.experimental.pallas import tpu as pltpu
from jax.experimental.pallas import tpu_sc as plsc

_N = 10000
_E = 320000
_D = 128
_NPAD = 10240
_BUCK = 320
_NW = 32
_EPT = 10112
_EPAD = _NW * _EPT
_CAP = 10752
_PK = 16384
_BND = 48
_HR = 672

_mesh = plsc.VectorSubcoreMesh(core_axis_name="c", subcore_axis_name="s")

_LANE0F = None


def _lane0(dtype):
    return (lax.iota(jnp.int32, 16) == 0).astype(dtype)


@functools.partial(
    pl.kernel,
    out_type=(
        jax.ShapeDtypeStruct((_NW * _CAP,), jnp.int32),
        jax.ShapeDtypeStruct((_NW * _BND,), jnp.int32),
        jax.ShapeDtypeStruct((_NW * _NPAD,), jnp.float32),
    ),
    mesh=_mesh,
    scratch_types=[
        pltpu.VMEM((_EPT + 16,), jnp.int32),
        pltpu.VMEM((_EPT + 16,), jnp.int32),
        pltpu.VMEM((80, 128), jnp.float32),
        pltpu.VMEM((80, 128), jnp.float32),
        pltpu.VMEM((84, 128), jnp.int32),
        pltpu.VMEM((_BND + 16,), jnp.int32),
        pltpu.VMEM((768,), jnp.int32),
        pltpu.VMEM((768,), jnp.float32),
    ],
)
def _setup_sort(src_hbm, dst_hbm, sorted_hbm, bnd_hbm, hist_hbm,
                srcv, dstv, hist2, cnt2, outb2, bndb, stgi, stgf):
    cid = lax.axis_index("c")
    sid = lax.axis_index("s")
    wid = sid * 2 + cid
    lane = lax.iota(jnp.int32, 16)

    pltpu.sync_copy(src_hbm.at[pl.ds(pl.multiple_of(wid * _EPT, 8), _EPT)],
                    srcv.at[pl.ds(0, _EPT)])
    pltpu.sync_copy(dst_hbm.at[pl.ds(pl.multiple_of(wid * _EPT, 8), _EPT)],
                    dstv.at[pl.ds(0, _EPT)])

    zf = jnp.zeros((16,), jnp.float32)
    zi = jnp.zeros((16,), jnp.int32)

    def zrows(r, _):
        for c in range(8):
            hist2[r, pl.ds(c * 16, 16)] = zf
            cnt2[r, pl.ds(c * 16, 16)] = zf
            outb2[r % 84, pl.ds(c * 16, 16)] = zi
        return 0
    lax.fori_loop(0, 84, zrows, 0)

    def p1(e, _):
        d = dstv[pl.ds(e, 16)][0]
        row = lax.shift_right_logical(d, 7)
        col = jnp.bitwise_and(d, 112)
        oh = jnp.maximum(1 - jnp.abs(
            lane - jnp.broadcast_to(jnp.bitwise_and(d, 15), (16,))), 0
            ).astype(jnp.float32)
        hist2[row, pl.ds(col, 16)] = hist2[row, pl.ds(col, 16)] + oh
        return 0
    lax.fori_loop(0, _EPT, p1, 0)

    def hout(blk, _):
        for g in range(6):
            for c in range(8):
                stgf[pl.ds(g * 128 + c * 16, 16)] = (
                    hist2[blk * 6 + g, pl.ds(c * 16, 16)])
        pltpu.sync_copy(
            stgf,
            hist_hbm.at[pl.ds(pl.multiple_of(wid * _NPAD + blk * 768, 8), 768)])
        return 0
    lax.fori_loop(0, 13, hout, 0)
    for g in range(2):
        for c in range(8):
            stgf[pl.ds(g * 128 + c * 16, 16)] = hist2[78 + g, pl.ds(c * 16, 16)]
    pltpu.sync_copy(stgf.at[pl.ds(0, 256)],
                    hist_hbm.at[pl.ds(pl.multiple_of(wid * _NPAD + 9984, 8), 256)])

    def bucket_loop(t, s):
        s = jnp.bitwise_and(s + 7, -8)
        bndb[pl.ds(t, 16)] = jnp.broadcast_to(s, (16,))

        def slice_loop(ss, s2):
            g = t * 20 + ss
            row = lax.shift_right_logical(g, 3)
            col = jnp.bitwise_and(g, 7) * 16
            hv = hist2[row, pl.ds(col, 16)]
            v = jnp.broadcast_to(s2, (16,)).astype(jnp.float32)
            run = s2
            for j in range(16):
                v = jnp.where(lane == j,
                              jnp.broadcast_to(run, (16,)).astype(jnp.float32), v)
                run = run + hv[j].astype(jnp.int32)
            cnt2[row, pl.ds(col, 16)] = v
            return run
        return lax.fori_loop(0, 20, slice_loop, s)

    send = lax.fori_loop(0, _NW, bucket_loop, jnp.int32(0))
    bndb[pl.ds(_NW, 16)] = jnp.broadcast_to(send, (16,))

    def p2(e, _):
        sv = srcv[pl.ds(e, 16)][0]
        d = dstv[pl.ds(e, 16)][0]
        row = lax.shift_right_logical(d, 7)
        col = jnp.bitwise_and(d, 112)
        d15 = jnp.bitwise_and(d, 15)
        cv = cnt2[row, pl.ds(col, 16)]
        pos_f = cv[15]
        for j in range(15):
            pos_f = jnp.where(d15 == j, cv[j], pos_f)
        pos = pos_f.astype(jnp.int32)
        oh = jnp.maximum(1 - jnp.abs(lane - jnp.broadcast_to(d15, (16,))), 0
                         ).astype(jnp.float32)
        cnt2[row, pl.ds(col, 16)] = cv + oh
        p = (_N - sv) * _PK + d
        prow = lax.shift_right_logical(pos, 7)
        pcol = jnp.bitwise_and(pos, 112)
        p15 = jnp.bitwise_and(pos, 15)
        ohp = jnp.maximum(1 - jnp.abs(lane - jnp.broadcast_to(p15, (16,))), 0)
        ov = outb2[prow, pl.ds(pcol, 16)]
        outb2[prow, pl.ds(pcol, 16)] = (
            ov * (1 - ohp) + jnp.broadcast_to(p, (16,)) * ohp)
        return 0
    lax.fori_loop(0, _EPT, p2, 0)

    def sout(blk, _):
        for g in range(6):
            for c in range(8):
                stgi[pl.ds(g * 128 + c * 16, 16)] = (
                    outb2[blk * 6 + g, pl.ds(c * 16, 16)])
        pltpu.sync_copy(
            stgi,
            sorted_hbm.at[pl.ds(pl.multiple_of(wid * _CAP + blk * 768, 8), 768)])
        return 0
    lax.fori_loop(0, 14, sout, 0)

    pltpu.sync_copy(bndb.at[pl.ds(0, _BND)],
                    bnd_hbm.at[pl.ds(pl.multiple_of(wid * _BND, 8), _BND)])


@functools.partial(
    pl.kernel,
    out_type=jax.ShapeDtypeStruct((_NPAD, _D), jnp.float32),
    mesh=_mesh,
    scratch_types=[
        pltpu.VMEM((_BUCK + 16, _D), jnp.float32),
        pltpu.VMEM((256, _D), jnp.float32),
        pltpu.VMEM((256 + 16,), jnp.int32),
        pltpu.VMEM((128,), jnp.int32),
        pltpu.VMEM((128,), jnp.int32),
        pltpu.VMEM((_NW * _BND + 16,), jnp.int32),
        pltpu.SemaphoreType.DMA,
    ],
)
def _layer_agg(hs_hbm, sorted_hbm, bnd_hbm, out_hbm,
               acc, rows_v, pbuf, idxa, idxb, bndv, sem):
    cid = lax.axis_index("c")
    sid = lax.axis_index("s")
    wid = sid * 2 + cid
    base = wid * _BUCK

    zf = jnp.zeros((16,), jnp.float32)

    def za(r, _):
        for j in range(_D // 16):
            acc[r, pl.ds(j * 16, 16)] = zf
        return 0
    lax.fori_loop(0, _BUCK + 16, za, 0)

    pltpu.sync_copy(bnd_hbm, bndv.at[pl.ds(0, _NW * _BND)])

    def seg_loop(s, _):
        a0 = pl.multiple_of(bndv[pl.ds(s * _BND + wid, 16)][0], 8)
        a1 = bndv[pl.ds(s * _BND + wid + 1, 16)][0]
        nblk = lax.shift_right_logical(a1 - a0 + 255, 8)

        def block(b, _):
            off = a0 + b * 256
            pltpu.sync_copy(
                sorted_hbm.at[pl.ds(pl.multiple_of(s * _CAP + off, 8), 256)],
                pbuf.at[pl.ds(0, 256)])
            for g in range(8):
                p16 = pbuf[pl.ds(g * 16, 16)]
                sidx = _N - lax.shift_right_logical(p16, 14)
                idxa[pl.ds(g * 16, 16)] = jnp.minimum(jnp.maximum(sidx, 0), _N)
            nloc = jnp.minimum(a1 - off, 256)
            ca = pltpu.async_copy(hs_hbm.at[idxa], rows_v.at[pl.ds(0, 128)], sem)

            @pl.when(nloc > 128)
            def _():
                for g in range(8):
                    p16 = pbuf[pl.ds(128 + g * 16, 16)]
                    sidx = _N - lax.shift_right_logical(p16, 14)
                    idxb[pl.ds(g * 16, 16)] = jnp.minimum(jnp.maximum(sidx, 0), _N)
                cb = pltpu.async_copy(hs_hbm.at[idxb],
                                      rows_v.at[pl.ds(128, 128)], sem)
                cb.wait()
            ca.wait()

            def edge(i, _):
                e = i * 2
                p = pbuf[pl.ds(e, 16)][0]
                q = pbuf[pl.ds(e + 1, 16)][0]
                d = jnp.bitwise_and(p, _PK - 1) - base
                dq = jnp.bitwise_and(q, _PK - 1) - base
                d2 = jnp.where(jnp.logical_or(d < 0, d >= _BUCK), _BUCK, d)
                dq2 = jnp.where(jnp.logical_or(dq < 0, dq >= _BUCK), _BUCK, dq)
                for j in range(_D // 16):
                    acc[d2, pl.ds(j * 16, 16)] = (
                        acc[d2, pl.ds(j * 16, 16)] + rows_v[e, pl.ds(j * 16, 16)])
                for j in range(_D // 16):
                    acc[dq2, pl.ds(j * 16, 16)] = (
                        acc[dq2, pl.ds(j * 16, 16)]
                        + rows_v[e + 1, pl.ds(j * 16, 16)])
                return 0
            lax.fori_loop(0, lax.shift_right_logical(nloc + 1, 1), edge, 0)
            return 0
        lax.fori_loop(0, nblk, block, 0)
        return 0
    lax.fori_loop(0, _NW, seg_loop, 0)

    pltpu.sync_copy(acc.at[pl.ds(0, _BUCK)], out_hbm.at[pl.ds(base, _BUCK)])


def _prep_body(hist_ref, x_ref, w_ref, hs_ref, dinv_ref):
    ones = jnp.ones((_NW, 1), jnp.float32)
    deg = lax.dot_general(hist_ref[:], ones, (((0,), (0,)), ((), ())),
                          preferred_element_type=jnp.float32)
    dinv = lax.rsqrt(deg + 1.0)
    dinv_ref[:] = dinv
    h = jnp.dot(x_ref[:], w_ref[:], preferred_element_type=jnp.float32)
    hs_ref[pl.ds(0, _N), :] = h * dinv[:_N]
    hs_ref[pl.ds(_N, _NPAD - _N), :] = jnp.zeros((_NPAD - _N, _D), jnp.float32)


def _mid_body(agg_ref, hs_ref, dinv_ref, g_ref, be_ref, w_ref, out_ref):
    dinv = dinv_ref[pl.ds(0, _N)]
    t = (agg_ref[pl.ds(0, _N), :] + hs_ref[pl.ds(0, _N), :]) * dinv
    m = jnp.mean(t, axis=0, keepdims=True)
    y0 = t - m
    v = jnp.mean(y0 * y0, axis=0, keepdims=True)
    y = jnp.maximum(y0 * lax.rsqrt(v + 1e-5) * g_ref[:] + be_ref[:], 0.0)
    hn = jnp.dot(y, w_ref[:], preferred_element_type=jnp.float32) * dinv
    wout = w_ref.shape[1]
    out_ref[pl.ds(0, _N), :] = hn
    out_ref[pl.ds(_N, _NPAD - _N), :] = jnp.zeros((_NPAD - _N, wout), jnp.float32)


def _final_body(agg_ref, hs_ref, dinv_ref, g_ref, be_ref, wr_ref, br_ref, out_ref):
    dinv = dinv_ref[pl.ds(0, _N)]
    t = (agg_ref[pl.ds(0, _N), :] + hs_ref[pl.ds(0, _N), :]) * dinv
    m = jnp.mean(t, axis=0, keepdims=True)
    y0 = t - m
    v = jnp.mean(y0 * y0, axis=0, keepdims=True)
    y = jnp.maximum(y0 * lax.rsqrt(v + 1e-5) * g_ref[:] + be_ref[:], 0.0)
    z = jnp.dot(y, wr_ref[:], preferred_element_type=jnp.float32) + br_ref[:]
    out_ref[:] = jax.nn.sigmoid(z)


def kernel(x, edge_index, W1, b1, g1, be1, W2, b2, g2, be2, W3, b3, g3, be3, Wr, br):
    del b1, b2, b3
    ei = edge_index.astype(jnp.int32)
    padlen = _EPAD - _E
    pad = jnp.full((padlen,), _N, jnp.int32)
    src = jnp.concatenate([ei[0], pad])
    dst = jnp.concatenate([ei[1], pad])

    sorted_p, bnd, hist = _setup_sort(src, dst)
    hist2d = hist.reshape(_NW, _NPAD)

    hs1, dinv = pl.pallas_call(
        _prep_body,
        out_shape=(
            jax.ShapeDtypeStruct((_NPAD, _D), jnp.float32),
            jax.ShapeDtypeStruct((_NPAD, 1), jnp.float32),
        ),
    )(hist2d, x, W1)

    agg1 = _layer_agg(hs1, sorted_p, bnd)

    hs2 = pl.pallas_call(
        _mid_body,
        out_shape=jax.ShapeDtypeStruct((_NPAD, _D), jnp.float32),
    )(agg1, hs1, dinv, g1.reshape(1, -1), be1.reshape(1, -1), W2)

    agg2 = _layer_agg(hs2, sorted_p, bnd)

    half = _D // 2
    W3p = jnp.pad(W3, ((0, 0), (0, _D - half)))
    g3p = jnp.pad(g3, (0, _D - half)).reshape(1, -1)
    be3p = jnp.pad(be3, (0, _D - half)).reshape(1, -1)
    Wrp = jnp.pad(Wr, ((0, _D - half), (0, 0)))

    hs3 = pl.pallas_call(
        _mid_body,
        out_shape=jax.ShapeDtypeStruct((_NPAD, _D), jnp.float32),
    )(agg2, hs2, dinv, g2.reshape(1, -1), be2.reshape(1, -1), W3p)

    agg3 = _layer_agg(hs3, sorted_p, bnd)

    out = pl.pallas_call(
        _final_body,
        out_shape=jax.ShapeDtypeStruct((_N, 1), jnp.float32),
    )(agg3, hs3, dinv, g3p, be3p, Wrp, br.reshape(1, 1))

    return out
